# baseline (speedup 1.0000x reference)
import functools

import jax
import jax.numpy as jnp
from jax import lax
from jax.experimental import pallas as pl
from jax.experimental.pallas import tpu as pltpu
from jax.experimental.pallas import tpu_sc as plsc

NUM_ROWS = 100000
NUM_COLS = 256
CHUNK_ROWS = 80
N_CHUNKS = NUM_ROWS // CHUNK_ROWS
N_WORKERS = 32
NBUF = 6
MAX_ITERS = -(-N_CHUNKS // N_WORKERS)
N_GROUPS = -(-MAX_ITERS // NBUF)

_mesh = plsc.VectorSubcoreMesh(core_axis_name="c", subcore_axis_name="s")


def _rows(c):
    return pl.ds(c * CHUNK_ROWS, CHUNK_ROWS)


@functools.partial(
    pl.kernel,
    out_type=jax.ShapeDtypeStruct((NUM_ROWS, NUM_COLS), jnp.float32),
    mesh=_mesh,
    scratch_types=[
        pltpu.VMEM_SHARED((16, NBUF, CHUNK_ROWS, NUM_COLS), jnp.float32),
        [pltpu.SemaphoreType.DMA] * NBUF,
        [pltpu.SemaphoreType.DMA] * NBUF,
    ],
)
def _sc_copy(x_hbm, o_hbm, shared, isems, osems):
    sid = lax.axis_index("s")
    wid = sid * 2 + lax.axis_index("c")
    bufs = tuple(shared.at[sid, b] for b in range(NBUF))

    for b in range(NBUF):
        pltpu.async_copy(x_hbm.at[_rows(wid + b * N_WORKERS)], bufs[b], isems[b])

    def group(g, carry):
        for b in range(NBUF):
            i = g * NBUF + b
            c = wid + i * N_WORKERS

            @pl.when(c < N_CHUNKS)
            def _(b=b, c=c):
                pltpu.make_async_copy(x_hbm.at[_rows(c)], bufs[b], isems[b]).wait()
                pltpu.async_copy(bufs[b], o_hbm.at[_rows(c)], osems[b])

            j = i - (NBUF - 1)
            bj = (b + 1) % NBUF
            cj = wid + j * N_WORKERS
            cn = cj + NBUF * N_WORKERS

            @pl.when(jnp.logical_and(j >= 0, cn < N_CHUNKS))
            def _(bj=bj, cj=cj, cn=cn):
                pltpu.make_async_copy(bufs[bj], o_hbm.at[_rows(cj)], osems[bj]).wait()
                pltpu.async_copy(x_hbm.at[_rows(cn)], bufs[bj], isems[bj])

        return carry

    lax.fori_loop(0, N_GROUPS, group, 0)

    for b in range(NBUF):
        pltpu.make_async_copy(bufs[b], o_hbm.at[_rows(0)], osems[b]).wait()


def kernel(weight):
    return _sc_copy(weight)


# speedup vs baseline: 1.0653x; 1.0653x over previous
import functools

import jax
import jax.numpy as jnp
from jax import lax
from jax.experimental import pallas as pl
from jax.experimental.pallas import tpu as pltpu
from jax.experimental.pallas import tpu_sc as plsc

NUM_ROWS = 100000
NUM_COLS = 256
CHUNK_ROWS = 200
N_CHUNKS = NUM_ROWS // CHUNK_ROWS
N_WORKERS = 32
NBUF = 2
MAX_ITERS = -(-N_CHUNKS // N_WORKERS)
N_GROUPS = -(-MAX_ITERS // NBUF)

_mesh = plsc.VectorSubcoreMesh(core_axis_name="c", subcore_axis_name="s")


def _rows(c):
    return pl.ds(c * CHUNK_ROWS, CHUNK_ROWS)


@functools.partial(
    pl.kernel,
    out_type=jax.ShapeDtypeStruct((NUM_ROWS, NUM_COLS), jnp.float32),
    mesh=_mesh,
    scratch_types=[
        pltpu.VMEM_SHARED((16, NBUF, CHUNK_ROWS, NUM_COLS), jnp.float32),
        [pltpu.SemaphoreType.DMA] * NBUF,
        [pltpu.SemaphoreType.DMA] * NBUF,
    ],
)
def _sc_copy(x_hbm, o_hbm, shared, isems, osems):
    sid = lax.axis_index("s")
    wid = sid * 2 + lax.axis_index("c")
    bufs = tuple(shared.at[sid, b] for b in range(NBUF))

    for b in range(NBUF):
        pltpu.async_copy(x_hbm.at[_rows(wid + b * N_WORKERS)], bufs[b], isems[b])

    def group(g, carry):
        for b in range(NBUF):
            i = g * NBUF + b
            c = wid + i * N_WORKERS

            @pl.when(c < N_CHUNKS)
            def _(b=b, c=c):
                pltpu.make_async_copy(x_hbm.at[_rows(c)], bufs[b], isems[b]).wait()
                pltpu.async_copy(bufs[b], o_hbm.at[_rows(c)], osems[b])

            j = i - (NBUF - 1)
            bj = (b + 1) % NBUF
            cj = wid + j * N_WORKERS
            cn = cj + NBUF * N_WORKERS

            @pl.when(jnp.logical_and(j >= 0, cn < N_CHUNKS))
            def _(bj=bj, cj=cj, cn=cn):
                pltpu.make_async_copy(bufs[bj], o_hbm.at[_rows(cj)], osems[bj]).wait()
                pltpu.async_copy(x_hbm.at[_rows(cn)], bufs[bj], isems[bj])

        return carry

    lax.fori_loop(0, N_GROUPS, group, 0)

    for b in range(NBUF):
        pltpu.make_async_copy(bufs[b], o_hbm.at[_rows(0)], osems[b]).wait()


def kernel(weight):
    return _sc_copy(weight)
